# initial kernel scaffold (unmeasured)
import jax
import jax.numpy as jnp
from jax import lax
from jax.experimental import pallas as pl
from jax.experimental.pallas import tpu as pltpu

N_DEV = 4
N_TOK = 2048
D = 1024
E_LOCAL = 8
CAP = 51
SLOTS = 64
GROWS = E_LOCAL * SLOTS
CHUNK = N_TOK // N_DEV


def kernel(x, router_W, route_idx, expert_W):
    del router_W

    def body(x_ref, idx_ref, w_ref, out_ref,
             acc_ref, y_ref, recv_ref, send_sems, recv_sems):
        p = lax.axis_index("i")
        right = lax.rem(p + 1, N_DEV)

        route = idx_ref[:, :]
        route_t = route.reshape(1, N_TOK)
        tok_i = lax.broadcasted_iota(jnp.int32, (N_TOK, N_TOK), 0)
        tok_j = lax.broadcasted_iota(jnp.int32, (N_TOK, N_TOK), 1)
        same = (route == route_t) & (tok_j <= tok_i)
        cnt = jnp.sum(same.astype(jnp.float32), axis=1, keepdims=True)
        cnt = cnt.astype(jnp.int32)
        le = route - E_LOCAL * p
        keep = (le >= 0) & (le < E_LOCAL) & (cnt <= CAP)
        target = jnp.where(keep, le * SLOTS + cnt - 1, -1)

        grow = lax.broadcasted_iota(jnp.int32, (GROWS, N_TOK), 0)
        S = (grow == target.reshape(1, N_TOK)).astype(jnp.float32)
        tcol = lax.broadcasted_iota(jnp.int32, (N_TOK, GROWS), 1)
        ST = (tcol == target).astype(jnp.float32)

        gathered = jnp.dot(S, x_ref[:, :], preferred_element_type=jnp.float32)
        for e in range(E_LOCAL):
            rows = slice(e * SLOTS, (e + 1) * SLOTS)
            y_ref[rows, :] = jnp.dot(
                gathered[rows, :], w_ref[e, :, :],
                preferred_element_type=jnp.float32)
        acc_ref[:, :] = jnp.dot(ST, y_ref[:, :],
                                preferred_element_type=jnp.float32)

        for s in range(N_DEV - 1):
            c_send = lax.rem(p - 1 - s + 2 * N_DEV, N_DEV)
            rdma = pltpu.make_async_remote_copy(
                src_ref=acc_ref.at[pl.ds(c_send * CHUNK, CHUNK)],
                dst_ref=recv_ref.at[s],
                send_sem=send_sems.at[s],
                recv_sem=recv_sems.at[s],
                device_id=(right,),
                device_id_type=pl.DeviceIdType.MESH,
            )
            rdma.start()
            rdma.wait()
            c_recv = lax.rem(p - 2 - s + 2 * N_DEV, N_DEV)
            rs = pl.ds(c_recv * CHUNK, CHUNK)
            acc_ref[rs, :] = acc_ref[rs, :] + recv_ref[s, :, :]

        out_ref[:, :] = acc_ref[pl.ds(p * CHUNK, CHUNK), :]

    return pl.pallas_call(
        body,
        out_shape=jax.ShapeDtypeStruct((CHUNK, D), jnp.float32),
        in_specs=[
            pl.BlockSpec(memory_space=pltpu.VMEM),
            pl.BlockSpec(memory_space=pltpu.VMEM),
            pl.BlockSpec(memory_space=pltpu.VMEM),
        ],
        out_specs=pl.BlockSpec(memory_space=pltpu.VMEM),
        scratch_shapes=[
            pltpu.VMEM((N_TOK, D), jnp.float32),
            pltpu.VMEM((GROWS, D), jnp.float32),
            pltpu.VMEM((N_DEV - 1, CHUNK, D), jnp.float32),
            pltpu.SemaphoreType.DMA((N_DEV - 1,)),
            pltpu.SemaphoreType.DMA((N_DEV - 1,)),
        ],
        compiler_params=pltpu.CompilerParams(collective_id=0),
    )(x, route_idx, expert_W)


# baseline (device time: 108111 ns/iter reference)
import jax
import jax.numpy as jnp
from jax import lax
from jax.experimental import pallas as pl
from jax.experimental.pallas import tpu as pltpu

N_DEV = 4
N_TOK = 2048
D = 1024
E_LOCAL = 8
CAP = 51
SLOTS = 64
GROWS = E_LOCAL * SLOTS
CHUNK = N_TOK // N_DEV


def kernel(x, router_W, route_idx, expert_W):
    del router_W

    def body(x_ref, idx_ref, w_ref, out_ref,
             acc_ref, y_ref, recv_ref, send_sems, recv_sems):
        p = lax.axis_index("i")
        right = lax.rem(p + 1, N_DEV)

        route = idx_ref[:, :]
        route_t = route.reshape(1, N_TOK)
        tok_i = lax.broadcasted_iota(jnp.int32, (N_TOK, N_TOK), 0)
        tok_j = lax.broadcasted_iota(jnp.int32, (N_TOK, N_TOK), 1)
        same = (route == route_t) & (tok_j <= tok_i)
        cnt = jnp.sum(same.astype(jnp.float32), axis=1, keepdims=True)
        cnt = cnt.astype(jnp.int32)
        le = route - E_LOCAL * p
        keep = (le >= 0) & (le < E_LOCAL) & (cnt <= CAP)
        target = jnp.where(keep, le * SLOTS + cnt - 1, -1)

        grow = lax.broadcasted_iota(jnp.int32, (GROWS, N_TOK), 0)
        S = (grow == target.reshape(1, N_TOK)).astype(jnp.float32)
        tcol = lax.broadcasted_iota(jnp.int32, (N_TOK, GROWS), 1)
        ST = (tcol == target).astype(jnp.float32)

        gathered = jnp.dot(S, x_ref[:, :], preferred_element_type=jnp.float32)
        for e in range(E_LOCAL):
            rows = slice(e * SLOTS, (e + 1) * SLOTS)
            y_ref[rows, :] = jnp.dot(
                gathered[rows, :], w_ref[e, :, :],
                preferred_element_type=jnp.float32)
        acc_ref[:, :] = jnp.dot(ST, y_ref[:, :],
                                preferred_element_type=jnp.float32)

        for s in range(N_DEV - 1):
            c_send = lax.rem(p - 1 - s + 2 * N_DEV, N_DEV)
            rdma = pltpu.make_async_remote_copy(
                src_ref=acc_ref.at[pl.ds(c_send * CHUNK, CHUNK)],
                dst_ref=recv_ref.at[s],
                send_sem=send_sems.at[s],
                recv_sem=recv_sems.at[s],
                device_id=(right,),
                device_id_type=pl.DeviceIdType.MESH,
            )
            rdma.start()
            rdma.wait()
            c_recv = lax.rem(p - 2 - s + 2 * N_DEV, N_DEV)
            rs = pl.ds(c_recv * CHUNK, CHUNK)
            acc_ref[rs, :] = acc_ref[rs, :] + recv_ref[s, :, :]

        out_ref[:, :] = acc_ref[pl.ds(p * CHUNK, CHUNK), :]

    return pl.pallas_call(
        body,
        out_shape=jax.ShapeDtypeStruct((CHUNK, D), jnp.float32),
        in_specs=[
            pl.BlockSpec(memory_space=pltpu.VMEM),
            pl.BlockSpec(memory_space=pltpu.VMEM),
            pl.BlockSpec(memory_space=pltpu.VMEM),
        ],
        out_specs=pl.BlockSpec(memory_space=pltpu.VMEM),
        scratch_shapes=[
            pltpu.VMEM((N_TOK, D), jnp.float32),
            pltpu.VMEM((GROWS, D), jnp.float32),
            pltpu.VMEM((N_DEV - 1, CHUNK, D), jnp.float32),
            pltpu.SemaphoreType.DMA((N_DEV - 1,)),
            pltpu.SemaphoreType.DMA((N_DEV - 1,)),
        ],
        compiler_params=pltpu.CompilerParams(
            vmem_limit_bytes=100 * 1024 * 1024,
        ),
    )(x, route_idx, expert_W)


# device time: 58714 ns/iter; 1.8413x vs baseline; 1.8413x over previous
import jax
import jax.numpy as jnp
from jax import lax
from jax.experimental import pallas as pl
from jax.experimental.pallas import tpu as pltpu

N_DEV = 4
N_TOK = 2048
D = 1024
E_LOCAL = 8
CAP = 51
SLOTS = 64
GROWS = E_LOCAL * SLOTS
CHUNK = N_TOK // N_DEV

_SEND_ORDER = (2, 1, 3)


def kernel(x, router_W, route_idx, expert_W):
    del router_W

    def body(x_ref, idx_ref, w_ref, out_ref,
             tgt_ref, y_ref, send_ref, recv_ref, send_sems, recv_sems):
        p = lax.axis_index("i")

        route = idx_ref[:, :]
        route_t = route.reshape(1, N_TOK)
        tok_i = lax.broadcasted_iota(jnp.int32, (N_TOK, N_TOK), 0)
        tok_j = lax.broadcasted_iota(jnp.int32, (N_TOK, N_TOK), 1)
        same = (route == route_t) & (tok_j <= tok_i)
        cnt = jnp.sum(same.astype(jnp.float32), axis=1, keepdims=True)
        cnt = cnt.astype(jnp.int32)
        le = route - E_LOCAL * p
        keep = (le >= 0) & (le < E_LOCAL) & (cnt <= CAP)
        target = jnp.where(keep, le * SLOTS + cnt - 1, -1)
        tgt_ref[:, :] = target

        grow = lax.broadcasted_iota(jnp.int32, (GROWS, N_TOK), 0)
        S = (grow == target.reshape(1, N_TOK)).astype(jnp.bfloat16)
        xb = x_ref[:, :].astype(jnp.bfloat16)
        gathered = jnp.dot(S, xb, preferred_element_type=jnp.float32)
        gathered = gathered.astype(jnp.bfloat16)
        for e in range(E_LOCAL):
            rows = slice(e * SLOTS, (e + 1) * SLOTS)
            y_ref[rows, :] = jnp.dot(
                gathered[rows, :], w_ref[e, :, :].astype(jnp.bfloat16),
                preferred_element_type=jnp.float32).astype(jnp.bfloat16)

        ycols = lax.broadcasted_iota(jnp.int32, (CHUNK, GROWS), 1)
        send_descs = []
        for si, m in enumerate(_SEND_ORDER):
            k = lax.rem(p + m, N_DEV)
            slot = (4 - m) % 4 - 1
            tsl = tgt_ref[pl.ds(k * CHUNK, CHUNK), :]
            STk = (ycols == tsl).astype(jnp.bfloat16)
            send_ref[si, :, :] = jnp.dot(
                STk, y_ref[:, :],
                preferred_element_type=jnp.float32).astype(jnp.bfloat16)
            rdma = pltpu.make_async_remote_copy(
                src_ref=send_ref.at[si],
                dst_ref=recv_ref.at[slot],
                send_sem=send_sems.at[si],
                recv_sem=recv_sems.at[slot],
                device_id=(k,),
                device_id_type=pl.DeviceIdType.MESH,
            )
            rdma.start()
            send_descs.append(rdma)

        tsl = tgt_ref[pl.ds(p * CHUNK, CHUNK), :]
        STown = (ycols == tsl).astype(jnp.bfloat16)
        acc = jnp.dot(STown, y_ref[:, :], preferred_element_type=jnp.float32)

        for slot in (1, 2, 0):
            recv_d = pltpu.make_async_remote_copy(
                src_ref=send_ref.at[0],
                dst_ref=recv_ref.at[slot],
                send_sem=send_sems.at[0],
                recv_sem=recv_sems.at[slot],
                device_id=(p,),
                device_id_type=pl.DeviceIdType.MESH,
            )
            recv_d.wait_recv()
            acc = acc + recv_ref[slot, :, :].astype(jnp.float32)
        out_ref[:, :] = acc

        for rdma in send_descs:
            rdma.wait_send()

    return pl.pallas_call(
        body,
        out_shape=jax.ShapeDtypeStruct((CHUNK, D), jnp.float32),
        in_specs=[
            pl.BlockSpec(memory_space=pltpu.VMEM),
            pl.BlockSpec(memory_space=pltpu.VMEM),
            pl.BlockSpec(memory_space=pltpu.VMEM),
        ],
        out_specs=pl.BlockSpec(memory_space=pltpu.VMEM),
        scratch_shapes=[
            pltpu.VMEM((N_TOK, 1), jnp.int32),
            pltpu.VMEM((GROWS, D), jnp.bfloat16),
            pltpu.VMEM((N_DEV - 1, CHUNK, D), jnp.bfloat16),
            pltpu.VMEM((N_DEV - 1, CHUNK, D), jnp.bfloat16),
            pltpu.SemaphoreType.DMA((N_DEV - 1,)),
            pltpu.SemaphoreType.DMA((N_DEV - 1,)),
        ],
        compiler_params=pltpu.CompilerParams(
            vmem_limit_bytes=100 * 1024 * 1024,
        ),
    )(x, route_idx, expert_W)


# device time: 53432 ns/iter; 2.0233x vs baseline; 1.0989x over previous
import jax
import jax.numpy as jnp
from jax import lax
from jax.experimental import pallas as pl
from jax.experimental.pallas import tpu as pltpu

N_DEV = 4
N_TOK = 2048
D = 1024
E_LOCAL = 8
CAP = 51
SLOTS = 64
GROWS = E_LOCAL * SLOTS
CHUNK = N_TOK // N_DEV
NG = 2
GR = GROWS // NG

_SEND_ORDER = (2, 1, 3)
_RECV_ORDER = (1, 2, 0)

_COMM = True
_COMPUTE = True


def kernel(x, router_W, route_idx, expert_W):
    del router_W

    def body(x_ref, idx_ref, w_ref, out_ref,
             cnt_ref, y_ref, recv_ref, send_sems, recv_sems):
        p = lax.axis_index("i")

        route = idx_ref[:, :]
        eids = lax.broadcasted_iota(jnp.int32, (N_TOK, 32), 1)
        onehot = (route == eids).astype(jnp.float32)
        tril = (lax.broadcasted_iota(jnp.int32, (128, 128), 0)
                >= lax.broadcasted_iota(jnp.int32, (128, 128), 1)
                ).astype(jnp.float32)
        carry = jnp.zeros((1, 32), jnp.float32)
        blocks = []
        for b in range(N_TOK // 128):
            local = jnp.dot(tril, onehot[b * 128:(b + 1) * 128, :],
                            preferred_element_type=jnp.float32)
            blocks.append(local + carry)
            carry = carry + local[127:128, :]
        csum = jnp.concatenate(blocks, axis=0)
        cnt = jnp.sum(csum * onehot, axis=1, keepdims=True)
        cnt = cnt.astype(jnp.int32)
        cnt_ref[:, :] = cnt

        le = route - E_LOCAL * p
        keep = (le >= 0) & (le < E_LOCAL) & (cnt <= CAP)
        target = jnp.where(keep, le * SLOTS + cnt - 1, -1)
        grow = lax.broadcasted_iota(jnp.int32, (GROWS, N_TOK), 0)
        S = (grow == target.reshape(1, N_TOK)).astype(jnp.bfloat16)
        xb = x_ref[:, :].astype(jnp.bfloat16)
        gathered = jnp.dot(S, xb, preferred_element_type=jnp.float32)
        gathered = gathered.astype(jnp.bfloat16)

        if _COMM:
            barrier_sem = pltpu.get_barrier_semaphore()
            for m in (1, 2, 3):
                nbr = lax.rem(p + m, N_DEV)
                pl.semaphore_signal(
                    barrier_sem, inc=1,
                    device_id=(nbr,), device_id_type=pl.DeviceIdType.MESH,
                )
            pl.semaphore_wait(barrier_sem, N_DEV - 1)

        send_descs = []
        for g in range(NG):
            for e in range(g * E_LOCAL // NG, (g + 1) * E_LOCAL // NG):
                rows = slice(e * SLOTS, (e + 1) * SLOTS)
                y_ref[rows, :] = jnp.dot(
                    gathered[rows, :], w_ref[e, :, :].astype(jnp.bfloat16),
                    preferred_element_type=jnp.float32).astype(jnp.bfloat16)
            if _COMM:
                for si, m in enumerate(_SEND_ORDER):
                    k = lax.rem(p + m, N_DEV)
                    slot = (4 - m) % 4 - 1
                    rdma = pltpu.make_async_remote_copy(
                        src_ref=y_ref.at[pl.ds(g * GR, GR), :],
                        dst_ref=recv_ref.at[slot, g],
                        send_sem=send_sems.at[si, g],
                        recv_sem=recv_sems.at[slot, g],
                        device_id=(k,),
                        device_id_type=pl.DeviceIdType.MESH,
                    )
                    rdma.start()
                    send_descs.append(rdma)

        route_c = idx_ref[pl.ds(p * CHUNK, CHUNK), :]
        cnt_c = cnt_ref[pl.ds(p * CHUNK, CHUNK), :]
        ycols = lax.broadcasted_iota(jnp.int32, (CHUNK, GROWS), 1)

        def st_for(q):
            le_q = route_c - E_LOCAL * q
            keep_q = (le_q >= 0) & (le_q < E_LOCAL) & (cnt_c <= CAP)
            t2 = jnp.where(keep_q, le_q * SLOTS + cnt_c - 1, -1)
            return (ycols == t2).astype(jnp.bfloat16)

        acc = jnp.dot(st_for(p), y_ref[:, :], preferred_element_type=jnp.float32)

        if _COMM:
            STr = [st_for(lax.rem(p + m, N_DEV)) for m in _SEND_ORDER]
            for g in range(NG):
                for si, slot in enumerate(_RECV_ORDER):
                    recv_d = pltpu.make_async_remote_copy(
                        src_ref=y_ref.at[pl.ds(g * GR, GR), :],
                        dst_ref=recv_ref.at[slot, g],
                        send_sem=send_sems.at[0, g],
                        recv_sem=recv_sems.at[slot, g],
                        device_id=(p,),
                        device_id_type=pl.DeviceIdType.MESH,
                    )
                    recv_d.wait_recv()
                    m_src = slot + 1
                    sti = _SEND_ORDER.index(m_src)
                    acc = acc + jnp.dot(
                        STr[sti][:, g * GR:(g + 1) * GR],
                        recv_ref[slot, g, :, :],
                        preferred_element_type=jnp.float32)
        out_ref[:, :] = acc

        for rdma in send_descs:
            rdma.wait_send()

    return pl.pallas_call(
        body,
        out_shape=jax.ShapeDtypeStruct((CHUNK, D), jnp.float32),
        in_specs=[
            pl.BlockSpec(memory_space=pltpu.VMEM),
            pl.BlockSpec(memory_space=pltpu.VMEM),
            pl.BlockSpec(memory_space=pltpu.VMEM),
        ],
        out_specs=pl.BlockSpec(memory_space=pltpu.VMEM),
        scratch_shapes=[
            pltpu.VMEM((N_TOK, 1), jnp.int32),
            pltpu.VMEM((GROWS, D), jnp.bfloat16),
            pltpu.VMEM((N_DEV - 1, NG, GR, D), jnp.bfloat16),
            pltpu.SemaphoreType.DMA((N_DEV - 1, NG)),
            pltpu.SemaphoreType.DMA((N_DEV - 1, NG)),
        ],
        compiler_params=pltpu.CompilerParams(
            vmem_limit_bytes=100 * 1024 * 1024,
            collective_id=0,
        ),
    )(x, route_idx, expert_W)
